# baseline (speedup 1.0000x reference)
import functools

import jax
import jax.numpy as jnp
from jax import lax
from jax.experimental import pallas as pl
from jax.experimental.pallas import tpu as pltpu
from jax.experimental.pallas import tpu_sc as plsc

B = 4
H = 224
W = 224
C = 192
OUT_H = 224
OUT_W = 224
NPIX = B * OUT_H * OUT_W

NW = 32
PW = NPIX // NW
K = 16
NCH = PW // K
CP = 256

_HB = 28
_NH = H // _HB


def _pool_body(x_ref, out_ref):
    h = pl.program_id(0)

    @pl.when(h == 0)
    def _():
        out_ref[...] = jnp.zeros_like(out_ref)

    out_ref[...] += jnp.sum(x_ref[...], axis=(1, 2))


def _bfr(v):
    return v.astype(jnp.bfloat16).astype(jnp.float32)


def _grid_body(pooled_ref, w_ref, b_ref, xl_ref, yl_ref,
               ia_ref, ib_ref, ic_ref, id_ref,
               wa_ref, wb_ref, wc_ref, wd_ref):
    bidx = pl.program_id(0)
    pooled = pooled_ref[...] * (1.0 / (H * W))
    theta_all = jnp.dot(pooled.astype(jnp.bfloat16),
                        w_ref[...].astype(jnp.bfloat16),
                        preferred_element_type=jnp.float32) + b_ref[...]
    rowmask = lax.broadcasted_iota(jnp.int32, (B, 6), 0) == bidx
    theta = _bfr(jnp.sum(jnp.where(rowmask, theta_all, 0.0), axis=0,
                         keepdims=True))

    def t(j):
        return lax.slice(theta, (0, j), (1, j + 1))

    xg = _bfr(xl_ref[...])
    yg = _bfr(yl_ref[...])
    x_s = t(0) * xg + t(1) * yg + t(2)
    y_s = t(3) * xg + t(4) * yg + t(5)
    x = 0.5 * (x_s + 1.0) * W
    y = 0.5 * (y_s + 1.0) * H
    x0 = x.astype(jnp.int32)
    x1 = x0 + 1
    y0 = y.astype(jnp.int32)
    y1 = y0 + 1
    x0 = jnp.clip(x0, 0, H - 1)
    x1 = jnp.clip(x1, 0, H - 1)
    y0 = jnp.clip(y0, 0, W - 1)
    y1 = jnp.clip(y1, 0, W - 1)
    base = bidx * (H * W)
    ia_ref[0] = base + y0 * W + x0
    ib_ref[0] = base + y1 * W + x0
    ic_ref[0] = base + y0 * W + x1
    id_ref[0] = base + y1 * W + x1
    x0f = x0.astype(jnp.float32)
    x1f = x1.astype(jnp.float32)
    y0f = y0.astype(jnp.float32)
    y1f = y1.astype(jnp.float32)
    wa_ref[0] = (x1f - x) * (y1f - y)
    wb_ref[0] = (x1f - x) * (y - y0f)
    wc_ref[0] = (x - x0f) * (y1f - y)
    wd_ref[0] = (x - x0f) * (y - y0f)


def _make_sc_kernel():
    mesh = plsc.VectorSubcoreMesh(core_axis_name="c", subcore_axis_name="s",
                                  num_cores=2, num_subcores=16)

    @functools.partial(
        pl.kernel,
        mesh=mesh,
        out_type=jax.ShapeDtypeStruct((NPIX, C), jnp.float32),
        scratch_types=[
            pltpu.VMEM((NCH * 4 * K,), jnp.int32),
            pltpu.VMEM((NCH * 4 * K,), jnp.float32),
            pltpu.VMEM((4, K, CP), jnp.float32),
            pltpu.VMEM((4, K, CP), jnp.float32),
            pltpu.VMEM((K, C), jnp.float32),
            pltpu.VMEM((K, C), jnp.float32),
            pltpu.SemaphoreType.DMA,
            pltpu.SemaphoreType.DMA,
            pltpu.SemaphoreType.DMA,
            pltpu.SemaphoreType.DMA,
        ],
    )
    def sc_kernel(table, idx, w, out,
                  idx_v, w_v, rows0, rows1, out_v0, out_v1,
                  gsem0, gsem1, osem0, osem1):
        wid = lax.axis_index("s") * 2 + lax.axis_index("c")
        base = wid * PW
        pltpu.sync_copy(idx.at[pl.ds(wid * (NCH * 4 * K), NCH * 4 * K)],
                        idx_v)
        pltpu.sync_copy(w.at[pl.ds(wid * (NCH * 4 * K), NCH * 4 * K)], w_v)

        def fire(j, rows, gsem):
            for c in range(4):
                pltpu.async_copy(
                    table.at[idx_v.at[pl.ds((j * 4 + c) * K, K)]],
                    rows.at[c], gsem)

        def drain(rows, gsem):
            for c in range(4):
                pltpu.make_async_copy(table.at[pl.ds(0, K)], rows.at[c],
                                      gsem).wait()

        def compute(j, rows, out_v):
            def grp_body(g, c2):
                woff = j * 4 * K + g * 16
                wav = w_v[pl.ds(woff, 16)]
                wbv = w_v[pl.ds(woff + K, 16)]
                wcv = w_v[pl.ds(woff + 2 * K, 16)]
                wdv = w_v[pl.ds(woff + 3 * K, 16)]
                dn = lax.GatherDimensionNumbers(
                    offset_dims=(), collapsed_slice_dims=(0,),
                    start_index_map=(0,))

                def splat(vec, jj):
                    sp = jnp.full((16, 1), jj, jnp.int32)
                    return lax.gather(
                        vec, sp, dn, slice_sizes=(1,),
                        mode=lax.GatherScatterMode.PROMISE_IN_BOUNDS)

                for jj in range(16):
                    i = g * 16 + jj
                    was = splat(wav, jj)
                    wbs = splat(wbv, jj)
                    wcs = splat(wcv, jj)
                    wds = splat(wdv, jj)
                    for blk in range(C // 16):
                        sl = pl.ds(blk * 16, 16)
                        out_v[i, sl] = (
                            was * rows[0, i, sl] + wbs * rows[1, i, sl]
                            + wcs * rows[2, i, sl] + wds * rows[3, i, sl])
                return c2

            lax.fori_loop(0, K // 16, grp_body, 0)

        fire(0, rows0, gsem0)
        fire(1, rows1, gsem1)

        def body(t, carry):
            for s, rows, out_v, gsem, osem in (
                    (0, rows0, out_v0, gsem0, osem0),
                    (1, rows1, out_v1, gsem1, osem1)):
                j = 2 * t + s
                drain(rows, gsem)

                @pl.when(t > 0)
                def _():
                    pltpu.make_async_copy(out_v, out.at[pl.ds(0, K)],
                                          osem).wait()

                compute(j, rows, out_v)
                pltpu.async_copy(out_v, out.at[pl.ds(base + j * K, K)],
                                 osem)

                @pl.when(t < NCH // 2 - 1)
                def _():
                    fire(j + 2, rows, gsem)
            return carry

        lax.fori_loop(0, NCH // 2, body, 0)
        pltpu.make_async_copy(out_v0, out.at[pl.ds(0, K)], osem0).wait()
        pltpu.make_async_copy(out_v1, out.at[pl.ds(0, K)], osem1).wait()

    return sc_kernel


@functools.cache
def _get_sc_kernel():
    return _make_sc_kernel()


def _prep(X, W_loc, b_loc):
    pooled_sum = pl.pallas_call(
        _pool_body,
        grid=(_NH,),
        in_specs=[pl.BlockSpec((B, _HB, W, C), lambda h: (0, h, 0, 0))],
        out_specs=pl.BlockSpec((B, C), lambda h: (0, 0)),
        out_shape=jax.ShapeDtypeStruct((B, C), jnp.float32),
    )(X)

    plane_i = jax.ShapeDtypeStruct((B, OUT_H, OUT_W), jnp.int32)
    plane_f = jax.ShapeDtypeStruct((B, OUT_H, OUT_W), jnp.float32)
    return pl.pallas_call(
        _grid_body,
        grid=(B,),
        in_specs=[
            pl.BlockSpec((B, C), lambda b: (0, 0)),
            pl.BlockSpec((C, 6), lambda b: (0, 0)),
            pl.BlockSpec((1, 6), lambda b: (0, 0)),
            pl.BlockSpec((1, OUT_W), lambda b: (0, 0)),
            pl.BlockSpec((OUT_H, 1), lambda b: (0, 0)),
        ],
        out_specs=[pl.BlockSpec((1, OUT_H, OUT_W), lambda b: (b, 0, 0))] * 8,
        out_shape=[plane_i] * 4 + [plane_f] * 4,
    )(pooled_sum, W_loc, b_loc.reshape(1, 6),
      jnp.linspace(-1.0, 1.0, OUT_W).reshape(1, OUT_W),
      jnp.linspace(-1.0, 1.0, OUT_H).reshape(OUT_H, 1))


def _chunk_major(parts):
    stacked = jnp.stack([p.reshape(NPIX) for p in parts], axis=0)
    return stacked.reshape(4, NPIX // K, K).transpose(1, 0, 2).reshape(-1)


def kernel(X, W_loc, b_loc):
    ia, ib, ic, idd, wa, wb, wc, wd = _prep(X, W_loc, b_loc)
    idx = _chunk_major((ia, ib, ic, idd))
    w = _chunk_major((wa, wb, wc, wd))
    tab = jnp.pad(X.reshape(NPIX, C), ((0, 0), (0, CP - C)))
    out_flat = _get_sc_kernel()(tab, idx, w)
    one = b_loc[1] * 0.0 + 1.0
    return out_flat.reshape(B, OUT_H, OUT_W, C) * one


# speedup vs baseline: 1.1444x; 1.1444x over previous
import functools

import jax
import jax.numpy as jnp
from jax import lax
from jax.experimental import pallas as pl
from jax.experimental.pallas import tpu as pltpu
from jax.experimental.pallas import tpu_sc as plsc

B = 4
H = 224
W = 224
C = 192
OUT_H = 224
OUT_W = 224
NPIX = B * OUT_H * OUT_W

NW = 32
PW = NPIX // NW
K = 16
NCH = PW // K

_HB = 28
_NH = H // _HB


def _pool_body(x_ref, out_ref):
    h = pl.program_id(0)

    @pl.when(h == 0)
    def _():
        out_ref[...] = jnp.zeros_like(out_ref)

    out_ref[...] += jnp.sum(x_ref[...], axis=(1, 2))


def _bfr(v):
    return v.astype(jnp.bfloat16).astype(jnp.float32)


def _grid_body(pooled_ref, w_ref, b_ref, xl_ref, yl_ref,
               ia_ref, ib_ref, ic_ref, id_ref,
               wa_ref, wb_ref, wc_ref, wd_ref):
    bidx = pl.program_id(0)
    pooled = pooled_ref[...] * (1.0 / (H * W))
    theta_all = jnp.dot(pooled.astype(jnp.bfloat16),
                        w_ref[...].astype(jnp.bfloat16),
                        preferred_element_type=jnp.float32) + b_ref[...]
    rowmask = lax.broadcasted_iota(jnp.int32, (B, 6), 0) == bidx
    theta = _bfr(jnp.sum(jnp.where(rowmask, theta_all, 0.0), axis=0,
                         keepdims=True))

    def t(j):
        return lax.slice(theta, (0, j), (1, j + 1))

    xg = _bfr(xl_ref[...])
    yg = _bfr(yl_ref[...])
    x_s = t(0) * xg + t(1) * yg + t(2)
    y_s = t(3) * xg + t(4) * yg + t(5)
    x = 0.5 * (x_s + 1.0) * W
    y = 0.5 * (y_s + 1.0) * H
    x0 = x.astype(jnp.int32)
    x1 = x0 + 1
    y0 = y.astype(jnp.int32)
    y1 = y0 + 1
    x0 = jnp.clip(x0, 0, H - 1)
    x1 = jnp.clip(x1, 0, H - 1)
    y0 = jnp.clip(y0, 0, W - 1)
    y1 = jnp.clip(y1, 0, W - 1)
    base = bidx * (H * W)
    ia_ref[0] = base + y0 * W + x0
    ib_ref[0] = base + y1 * W + x0
    ic_ref[0] = base + y0 * W + x1
    id_ref[0] = base + y1 * W + x1
    x0f = x0.astype(jnp.float32)
    x1f = x1.astype(jnp.float32)
    y0f = y0.astype(jnp.float32)
    y1f = y1.astype(jnp.float32)
    wa_ref[0] = (x1f - x) * (y1f - y)
    wb_ref[0] = (x1f - x) * (y - y0f)
    wc_ref[0] = (x - x0f) * (y1f - y)
    wd_ref[0] = (x - x0f) * (y - y0f)


def _make_sc_kernel():
    mesh = plsc.VectorSubcoreMesh(core_axis_name="c", subcore_axis_name="s",
                                  num_cores=2, num_subcores=16)

    @functools.partial(
        pl.kernel,
        mesh=mesh,
        out_type=jax.ShapeDtypeStruct((NPIX, C), jnp.float32),
        compiler_params=pltpu.CompilerParams(use_tc_tiling_on_sc=False),
        scratch_types=[
            pltpu.VMEM((NCH * 4 * K,), jnp.int32),
            pltpu.VMEM((4 * K,), jnp.float32),
            pltpu.VMEM((4 * K,), jnp.float32),
            pltpu.VMEM((4, K, C), jnp.float32),
            pltpu.VMEM((4, K, C), jnp.float32),
            pltpu.VMEM((K, C), jnp.float32),
            pltpu.VMEM((K, C), jnp.float32),
            pltpu.SemaphoreType.DMA,
            pltpu.SemaphoreType.DMA,
            pltpu.SemaphoreType.DMA,
            pltpu.SemaphoreType.DMA,
        ],
    )
    def sc_kernel(table, idx, w, out,
                  idx_v, wch0, wch1, rows0, rows1, out_v0, out_v1,
                  gsem0, gsem1, osem0, osem1):
        wid = lax.axis_index("s") * 2 + lax.axis_index("c")
        base = wid * PW
        wbase = wid * (NCH * 4 * K)
        pltpu.sync_copy(idx.at[pl.ds(wbase, NCH * 4 * K)], idx_v)

        def fire(j, rows, wch, gsem):
            for c in range(4):
                pltpu.async_copy(
                    table.at[idx_v.at[pl.ds((j * 4 + c) * K, K)]],
                    rows.at[c], gsem)
            pltpu.async_copy(w.at[pl.ds(wbase + j * 4 * K, 4 * K)], wch,
                             gsem)

        def drain(rows, wch, gsem):
            for c in range(4):
                pltpu.make_async_copy(table.at[pl.ds(0, K)], rows.at[c],
                                      gsem).wait()
            pltpu.make_async_copy(w.at[pl.ds(0, 4 * K)], wch, gsem).wait()

        dn = lax.GatherDimensionNumbers(
            offset_dims=(), collapsed_slice_dims=(0,),
            start_index_map=(0,))

        def splat(vec, jj):
            sp = jnp.full((16, 1), jj, jnp.int32)
            return lax.gather(vec, sp, dn, slice_sizes=(1,),
                              mode=lax.GatherScatterMode.PROMISE_IN_BOUNDS)

        def compute(rows, wch, out_v):
            wav = wch[pl.ds(0, 16)]
            wbv = wch[pl.ds(K, 16)]
            wcv = wch[pl.ds(2 * K, 16)]
            wdv = wch[pl.ds(3 * K, 16)]

            @plsc.parallel_loop(0, K, 1, unroll=2)
            def _px(jj):
                was = splat(wav, jj)
                wbs = splat(wbv, jj)
                wcs = splat(wcv, jj)
                wds = splat(wdv, jj)
                for blk in range(C // 16):
                    sl = pl.ds(blk * 16, 16)
                    out_v[jj, sl] = (
                        was * rows[0, jj, sl] + wbs * rows[1, jj, sl]
                        + wcs * rows[2, jj, sl] + wds * rows[3, jj, sl])

        fire(0, rows0, wch0, gsem0)
        fire(1, rows1, wch1, gsem1)

        def body(t, carry):
            for s, rows, wch, out_v, gsem, osem in (
                    (0, rows0, wch0, out_v0, gsem0, osem0),
                    (1, rows1, wch1, out_v1, gsem1, osem1)):
                j = 2 * t + s
                drain(rows, wch, gsem)

                @pl.when(t > 0)
                def _():
                    pltpu.make_async_copy(out_v, out.at[pl.ds(0, K)],
                                          osem).wait()

                compute(rows, wch, out_v)
                pltpu.async_copy(out_v, out.at[pl.ds(base + j * K, K)],
                                 osem)

                @pl.when(t < NCH // 2 - 1)
                def _():
                    fire(j + 2, rows, wch, gsem)
            return carry

        lax.fori_loop(0, NCH // 2, body, 0)
        pltpu.make_async_copy(out_v0, out.at[pl.ds(0, K)], osem0).wait()
        pltpu.make_async_copy(out_v1, out.at[pl.ds(0, K)], osem1).wait()

    return sc_kernel


@functools.cache
def _get_sc_kernel():
    return _make_sc_kernel()


def _prep(X, W_loc, b_loc):
    pooled_sum = pl.pallas_call(
        _pool_body,
        grid=(_NH,),
        in_specs=[pl.BlockSpec((B, _HB, W, C), lambda h: (0, h, 0, 0))],
        out_specs=pl.BlockSpec((B, C), lambda h: (0, 0)),
        out_shape=jax.ShapeDtypeStruct((B, C), jnp.float32),
    )(X)

    plane_i = jax.ShapeDtypeStruct((B, OUT_H, OUT_W), jnp.int32)
    plane_f = jax.ShapeDtypeStruct((B, OUT_H, OUT_W), jnp.float32)
    return pl.pallas_call(
        _grid_body,
        grid=(B,),
        in_specs=[
            pl.BlockSpec((B, C), lambda b: (0, 0)),
            pl.BlockSpec((C, 6), lambda b: (0, 0)),
            pl.BlockSpec((1, 6), lambda b: (0, 0)),
            pl.BlockSpec((1, OUT_W), lambda b: (0, 0)),
            pl.BlockSpec((OUT_H, 1), lambda b: (0, 0)),
        ],
        out_specs=[pl.BlockSpec((1, OUT_H, OUT_W), lambda b: (b, 0, 0))] * 8,
        out_shape=[plane_i] * 4 + [plane_f] * 4,
    )(pooled_sum, W_loc, b_loc.reshape(1, 6),
      jnp.linspace(-1.0, 1.0, OUT_W).reshape(1, OUT_W),
      jnp.linspace(-1.0, 1.0, OUT_H).reshape(OUT_H, 1))


def _chunk_major(parts):
    stacked = jnp.stack([p.reshape(NPIX) for p in parts], axis=0)
    return stacked.reshape(4, NPIX // K, K).transpose(1, 0, 2).reshape(-1)


def kernel(X, W_loc, b_loc):
    ia, ib, ic, idd, wa, wb, wc, wd = _prep(X, W_loc, b_loc)
    idx = _chunk_major((ia, ib, ic, idd))
    w = _chunk_major((wa, wb, wc, wd))
    out_flat = _get_sc_kernel()(X.reshape(NPIX, C), idx, w)
    return out_flat.reshape(B, OUT_H, OUT_W, C)
